# baseline (speedup 1.0000x reference)
import functools

import jax
import jax.numpy as jnp
from jax import lax
from jax.experimental import pallas as pl
from jax.experimental.pallas import tpu as pltpu
from jax.experimental.pallas import tpu_sc as plsc

_Q = 1000000
_D = 64
_SALT = 1000003
_NC = 2
_NS = 16
_NW = _NC * _NS
_CH = 32
_NBUF = 3
_WR = 128


def kernel(inputs, table):
    nf, b, _ = inputs.shape
    n = nf * b
    per_w = n // _NW
    nchunk = per_w // _CH
    agg = _WR // _CH

    flat_in = inputs.reshape(n).astype(jnp.int32)
    table3 = table.reshape(_Q // 8, 8, _D)

    mesh = plsc.VectorSubcoreMesh(core_axis_name="c", subcore_axis_name="s")

    @functools.partial(
        pl.kernel,
        out_type=jax.ShapeDtypeStruct((nf, _D, b), jnp.float32),
        mesh=mesh,
        compiler_params=pltpu.CompilerParams(needs_layout_passes=False),
        scratch_types=[
            pltpu.VMEM((per_w,), jnp.int32),
            pltpu.VMEM((per_w,), jnp.int32),
            pltpu.VMEM((_NBUF, _CH, 8, _D), jnp.float32),
            pltpu.VMEM((_D, _WR), jnp.float32),
            pltpu.SemaphoreType.DMA((_NBUF,)),
        ],
    )
    def sc_kernel(in_hbm, table_hbm, out_hbm, raw_ref, idx_ref, grp_ref,
                  rows_ref, gsem):
        wid = lax.axis_index("s") * _NC + lax.axis_index("c")
        wbase = wid * per_w

        pltpu.sync_copy(in_hbm.at[pl.ds(wbase, per_w)], raw_ref)

        def issue_chunk(r):
            f = (wbase + r * _CH) // b
            salt = (f + 1) * _SALT
            bb = lax.rem(r, _NBUF)

            def issue_vec(c, _):
                v = raw_ref[pl.ds(r * _CH + c * 16, 16)]
                vidx = (v + salt) % _Q
                idx_ref[pl.ds(r * _CH + c * 16, 16)] = vidx
                for jj in range(16):
                    g = vidx[jj] // 8
                    pltpu.make_async_copy(
                        table_hbm.at[pl.ds(g, 1)],
                        grp_ref.at[bb].at[pl.ds(c * 16 + jj, 1)],
                        gsem.at[bb]).start()
                return 0

            lax.fori_loop(0, _CH // 16, issue_vec, 0, unroll=True)

        def drain_chunk(r):
            bb = lax.rem(r, _NBUF)
            pltpu.make_async_copy(
                table_hbm.at[pl.ds(0, _CH)], grp_ref.at[bb],
                gsem.at[bb]).wait()

        def select_chunk(r):
            bb = lax.rem(r, _NBUF)
            col0 = lax.rem(r, agg) * _CH
            bb_v = jnp.full((16,), 0, jnp.int32) + bb
            j_iota = lax.iota(jnp.int32, 16)

            def sel_vec(c, _):
                s_v = lax.rem(idx_ref[pl.ds(r * _CH + c * 16, 16)], 8)
                j_v = j_iota + c * 16
                for d in range(_D):
                    d_v = jnp.full((16,), d, jnp.int32)
                    val = plsc.load_gather(grp_ref, [bb_v, j_v, s_v, d_v])
                    rows_ref[d, pl.ds(col0 + c * 16, 16)] = val
                return 0

            lax.fori_loop(0, _CH // 16, sel_vec, 0)

        def prologue(r, _):
            issue_chunk(r)
            return 0

        lax.fori_loop(0, _NBUF, prologue, 0)

        def main(r, _):
            drain_chunk(r)
            select_chunk(r)

            @pl.when(r + _NBUF < nchunk)
            def _():
                issue_chunk(r + _NBUF)

            @pl.when(lax.rem(r, agg) == agg - 1)
            def _():
                flat0 = wbase + (r - (agg - 1)) * _CH
                f = flat0 // b
                b0 = pl.multiple_of(flat0 - f * b, _WR)
                pltpu.sync_copy(rows_ref,
                                out_hbm.at[f].at[:, pl.ds(b0, _WR)])

            return 0

        lax.fori_loop(0, nchunk, main, 0)

    out = sc_kernel(flat_in, table3)
    return out.transpose(0, 2, 1)


# speedup vs baseline: 1.0048x; 1.0048x over previous
import functools

import jax
import jax.numpy as jnp
from jax import lax
from jax.experimental import pallas as pl
from jax.experimental.pallas import tpu as pltpu
from jax.experimental.pallas import tpu_sc as plsc

_Q = 1000000
_D = 64
_SALT = 1000003
_NC = 2
_NS = 16
_NW = _NC * _NS
_CH = 32
_NBUF = 3
_WR = 256


def kernel(inputs, table):
    nf, b, _ = inputs.shape
    n = nf * b
    per_w = n // _NW
    nchunk = per_w // _CH
    agg = _WR // _CH

    flat_in = inputs.reshape(n).astype(jnp.int32)
    table3 = table.reshape(_Q // 8, 8, _D)

    mesh = plsc.VectorSubcoreMesh(core_axis_name="c", subcore_axis_name="s")

    @functools.partial(
        pl.kernel,
        out_type=jax.ShapeDtypeStruct((nf, _D, b), jnp.float32),
        mesh=mesh,
        compiler_params=pltpu.CompilerParams(needs_layout_passes=False),
        scratch_types=[
            pltpu.VMEM((per_w,), jnp.int32),
            pltpu.VMEM((per_w,), jnp.int32),
            pltpu.VMEM((_NBUF, _CH, 8, _D), jnp.float32),
            pltpu.VMEM((_D, _WR), jnp.float32),
            pltpu.SemaphoreType.DMA((_NBUF,)),
        ],
    )
    def sc_kernel(in_hbm, table_hbm, out_hbm, raw_ref, idx_ref, grp_ref,
                  rows_ref, gsem):
        wid = lax.axis_index("s") * _NC + lax.axis_index("c")
        wbase = wid * per_w

        pltpu.sync_copy(in_hbm.at[pl.ds(wbase, per_w)], raw_ref)

        def issue_chunk(r):
            f = (wbase + r * _CH) // b
            salt = (f + 1) * _SALT
            bb = lax.rem(r, _NBUF)

            def issue_vec(c, _):
                v = raw_ref[pl.ds(r * _CH + c * 16, 16)]
                vidx = (v + salt) % _Q
                idx_ref[pl.ds(r * _CH + c * 16, 16)] = vidx
                for jj in range(16):
                    g = vidx[jj] // 8
                    pltpu.make_async_copy(
                        table_hbm.at[pl.ds(g, 1)],
                        grp_ref.at[bb].at[pl.ds(c * 16 + jj, 1)],
                        gsem.at[bb]).start()
                return 0

            lax.fori_loop(0, _CH // 16, issue_vec, 0, unroll=True)

        def drain_chunk(r):
            bb = lax.rem(r, _NBUF)
            pltpu.make_async_copy(
                table_hbm.at[pl.ds(0, _CH)], grp_ref.at[bb],
                gsem.at[bb]).wait()

        def select_chunk(r):
            bb = lax.rem(r, _NBUF)
            col0 = lax.rem(r, agg) * _CH
            bb_v = jnp.full((16,), 0, jnp.int32) + bb
            j_iota = lax.iota(jnp.int32, 16)

            def sel_vec(c, _):
                s_v = lax.rem(idx_ref[pl.ds(r * _CH + c * 16, 16)], 8)
                j_v = j_iota + c * 16
                for d in range(_D):
                    d_v = jnp.full((16,), d, jnp.int32)
                    val = plsc.load_gather(grp_ref, [bb_v, j_v, s_v, d_v])
                    rows_ref[d, pl.ds(col0 + c * 16, 16)] = val
                return 0

            lax.fori_loop(0, _CH // 16, sel_vec, 0)

        def prologue(r, _):
            issue_chunk(r)
            return 0

        lax.fori_loop(0, _NBUF, prologue, 0)

        def main(r, _):
            drain_chunk(r)
            select_chunk(r)

            @pl.when(r + _NBUF < nchunk)
            def _():
                issue_chunk(r + _NBUF)

            @pl.when(lax.rem(r, agg) == agg - 1)
            def _():
                flat0 = wbase + (r - (agg - 1)) * _CH
                f = flat0 // b
                b0 = pl.multiple_of(flat0 - f * b, _WR)
                pltpu.sync_copy(rows_ref,
                                out_hbm.at[f].at[:, pl.ds(b0, _WR)])

            return 0

        lax.fori_loop(0, nchunk, main, 0)

    out = sc_kernel(flat_in, table3)
    return out.transpose(0, 2, 1)
